# Initial kernel scaffold; baseline (speedup 1.0000x reference)
import functools

import jax
import jax.numpy as jnp
from jax import lax
from jax.experimental import pallas as pl
from jax.experimental.pallas import tpu as pltpu
from jax.experimental.pallas import tpu_sc as plsc

N_NODES = 10000
D = 128
N_EDGES = 320000

NC = 2
NS = 16
NW = NC * NS
CHUNK = 128
E_PAD = 327680
EPW = E_PAD // NW
N_CHUNKS = EPW // CHUNK
ROWS_PER_TILE = N_NODES // NS
ZROWS = 125


def _linear_body(x_ref, w_ref, b_ref, o_ref):
    o_ref[...] = lax.dot_general(
        x_ref[...], w_ref[...], (((1,), (1,)), ((), ())),
        preferred_element_type=jnp.float32) + b_ref[...]


def _tc_linear(x, W, b):
    return pl.pallas_call(
        _linear_body,
        grid=(10,),
        in_specs=[
            pl.BlockSpec((1000, D), lambda i: (i, 0)),
            pl.BlockSpec((D, D), lambda i: (0, 0)),
            pl.BlockSpec((1, D), lambda i: (0, 0)),
        ],
        out_specs=pl.BlockSpec((1000, D), lambda i: (i, 0)),
        out_shape=jax.ShapeDtypeStruct((N_NODES, D), jnp.float32),
    )(x, W, b.reshape(1, D))


def _combine_body(a0_ref, a1_ref, n0_ref, n1_ref, x_ref, w_ref, b_ref, o_ref):
    norm = n0_ref[...] + n1_ref[...]
    acc = a0_ref[...] + a1_ref[...]
    selfh = lax.dot_general(
        x_ref[...], w_ref[...], (((1,), (1,)), ((), ())),
        preferred_element_type=jnp.float32) + b_ref[...]
    o_ref[...] = acc / norm[:, 0:1] + selfh


def _tc_combine(a0, a1, n0, n1, x, W_self, b_self):
    return pl.pallas_call(
        _combine_body,
        grid=(10,),
        in_specs=[
            pl.BlockSpec((1000, D), lambda i: (i, 0)),
            pl.BlockSpec((1000, D), lambda i: (i, 0)),
            pl.BlockSpec((1000, 16), lambda i: (i, 0)),
            pl.BlockSpec((1000, 16), lambda i: (i, 0)),
            pl.BlockSpec((1000, D), lambda i: (i, 0)),
            pl.BlockSpec((D, D), lambda i: (0, 0)),
            pl.BlockSpec((1, D), lambda i: (0, 0)),
        ],
        out_specs=pl.BlockSpec((1000, D), lambda i: (i, 0)),
        out_shape=jax.ShapeDtypeStruct((N_NODES, D), jnp.float32),
    )(a0, a1, n0, n1, x, W_self, b_self.reshape(1, D))


def _sc_body(h_hbm, src_hbm, dst_hbm, adj_hbm,
             acc_out, norm_out,
             acc_sh, norm_sh,
             srcv, dstv, adjv, rows, normrow, zbuf, zbufn, sem):
    c = lax.axis_index("c")
    s = lax.axis_index("s")
    wid = c * NS + s

    z16 = jnp.zeros((16,), jnp.float32)

    def _zero_zbuf(r, _):
        for k in range(D // 16):
            zbuf[r, pl.ds(k * 16, 16)] = z16
        return 0
    lax.fori_loop(0, ZROWS, _zero_zbuf, 0)

    def _zero_zbufn(r, _):
        zbufn[r, :] = z16
        return 0
    lax.fori_loop(0, ROWS_PER_TILE, _zero_zbufn, 0)

    row0 = s * ROWS_PER_TILE
    for k in range(ROWS_PER_TILE // ZROWS):
        pltpu.sync_copy(zbuf, acc_sh.at[pl.ds(row0 + k * ZROWS, ZROWS)])
    pltpu.sync_copy(zbufn, norm_sh.at[pl.ds(row0, ROWS_PER_TILE)])

    def _zero_normrow(r, _):
        normrow[r, :] = z16
        return 0
    lax.fori_loop(0, CHUNK, _zero_normrow, 0)

    plsc.subcore_barrier()

    ebase = wid * EPW
    iota16 = lax.iota(jnp.int32, 16)
    zcol = jnp.zeros((16,), jnp.int32)

    def _chunk(ci, _):
        base = ebase + ci * CHUNK
        pltpu.sync_copy(src_hbm.at[pl.ds(base, CHUNK)], srcv)
        pltpu.sync_copy(dst_hbm.at[pl.ds(base, CHUNK)], dstv)
        pltpu.sync_copy(adj_hbm.at[pl.ds(base, CHUNK)], adjv)
        pltpu.async_copy(h_hbm.at[srcv], rows, sem).wait()

        def _scale(e, _):
            a = jnp.full((16,), adjv[e])
            for k in range(D // 16):
                rows[e, pl.ds(k * 16, 16)] = rows[e, pl.ds(k * 16, 16)] * a
            return 0
        lax.fori_loop(0, CHUNK, _scale, 0)

        def _nr(g, _):
            ridx = g * 16 + iota16
            plsc.store_scatter(normrow, [ridx, zcol], adjv[pl.ds(g * 16, 16)])
            return 0
        lax.fori_loop(0, CHUNK // 16, _nr, 0)

        pltpu.sync_copy(rows, acc_sh.at[dstv], add=True)
        pltpu.sync_copy(normrow, norm_sh.at[dstv], add=True)
        return 0

    lax.fori_loop(0, N_CHUNKS, _chunk, 0)

    plsc.subcore_barrier()

    pltpu.sync_copy(acc_sh.at[pl.ds(row0, ROWS_PER_TILE)],
                    acc_out.at[c, pl.ds(row0, ROWS_PER_TILE)])
    pltpu.sync_copy(norm_sh.at[pl.ds(row0, ROWS_PER_TILE)],
                    norm_out.at[c, pl.ds(row0, ROWS_PER_TILE)])


def _sc_scatter(h, src, dst, adj):
    mesh = plsc.VectorSubcoreMesh(core_axis_name="c", subcore_axis_name="s")
    f = pl.kernel(
        _sc_body,
        out_type=(
            jax.ShapeDtypeStruct((NC, N_NODES, D), jnp.float32),
            jax.ShapeDtypeStruct((NC, N_NODES, 16), jnp.float32),
        ),
        mesh=mesh,
        scratch_types=[
            pltpu.VMEM_SHARED((N_NODES, D), jnp.float32),
            pltpu.VMEM_SHARED((N_NODES, 16), jnp.float32),
            pltpu.VMEM((CHUNK,), jnp.int32),
            pltpu.VMEM((CHUNK,), jnp.int32),
            pltpu.VMEM((CHUNK,), jnp.float32),
            pltpu.VMEM((CHUNK, D), jnp.float32),
            pltpu.VMEM((CHUNK, 16), jnp.float32),
            pltpu.VMEM((ZROWS, D), jnp.float32),
            pltpu.VMEM((ROWS_PER_TILE, 16), jnp.float32),
            pltpu.SemaphoreType.DMA,
        ],
    )
    return f(h, src, dst, adj)


def kernel(node_feat, edge_index, adj_values, W, b, W_self, b_self):
    x = node_feat[0]
    dst = edge_index[0].astype(jnp.int32)
    src = edge_index[1].astype(jnp.int32)
    pad = E_PAD - N_EDGES
    src_p = jnp.concatenate([src, jnp.zeros((pad,), jnp.int32)])
    dst_p = jnp.concatenate([dst, jnp.zeros((pad,), jnp.int32)])
    adj_p = jnp.concatenate([adj_values, jnp.zeros((pad,), jnp.float32)])

    h = _tc_linear(x, W, b)
    acc, normp = _sc_scatter(h, src_p, dst_p, adj_p)
    out = _tc_combine(acc[0], acc[1], normp[0], normp[1], x, W_self, b_self)
    return out[None]


# speedup vs baseline: 2.6696x; 2.6696x over previous
import jax
import jax.numpy as jnp
from jax import lax
from jax.experimental import pallas as pl
from jax.experimental.pallas import tpu as pltpu
from jax.experimental.pallas import tpu_sc as plsc

N_NODES = 10000
D = 128
N_EDGES = 320000

NC = 2
NS = 16
NW = NC * NS
CHUNK = 128
E_PAD = 327680
EPW = E_PAD // NW
N_CHUNKS = EPW // CHUNK
N_PAD = 10240
ROWS_PER_TILE = N_PAD // NS
ZROWS = 128
BLK = 1024


def _linear_body(x_ref, w_ref, b_ref, o_ref):
    o_ref[...] = lax.dot_general(
        x_ref[...], w_ref[...], (((1,), (1,)), ((), ())),
        preferred_element_type=jnp.float32) + b_ref[...]


def _tc_linear(x, W, b):
    return pl.pallas_call(
        _linear_body,
        grid=(10,),
        in_specs=[
            pl.BlockSpec((1000, D), lambda i: (i, 0)),
            pl.BlockSpec((D, D), lambda i: (0, 0)),
            pl.BlockSpec((1, D), lambda i: (0, 0)),
        ],
        out_specs=pl.BlockSpec((1000, D), lambda i: (i, 0)),
        out_shape=jax.ShapeDtypeStruct((N_NODES, D), jnp.float32),
    )(x, W, b.reshape(1, D))


def _combine_body(a0_ref, a1_ref, n0_ref, n1_ref, x_ref, w_ref, b_ref, o_ref):
    norm = n0_ref[...][:, 0:1] + n1_ref[...][:, 0:1]
    acc = a0_ref[...] + a1_ref[...]
    selfh = lax.dot_general(
        x_ref[...], w_ref[...], (((1,), (1,)), ((), ())),
        preferred_element_type=jnp.float32) + b_ref[...]
    o_ref[...] = acc / norm + selfh


def _tc_combine(a0, a1, n0, n1, x_pad, W_self, b_self):
    return pl.pallas_call(
        _combine_body,
        grid=(10,),
        in_specs=[
            pl.BlockSpec((BLK, D), lambda i: (i, 0)),
            pl.BlockSpec((BLK, D), lambda i: (i, 0)),
            pl.BlockSpec((BLK, D), lambda i: (i, 0)),
            pl.BlockSpec((BLK, D), lambda i: (i, 0)),
            pl.BlockSpec((BLK, D), lambda i: (i, 0)),
            pl.BlockSpec((D, D), lambda i: (0, 0)),
            pl.BlockSpec((1, D), lambda i: (0, 0)),
        ],
        out_specs=pl.BlockSpec((BLK, D), lambda i: (i, 0)),
        out_shape=jax.ShapeDtypeStruct((N_PAD, D), jnp.float32),
    )(a0, a1, n0, n1, x_pad, W_self, b_self.reshape(1, D))


def _sc_feat_body(h_hbm, src_hbm, dst_hbm, adj_hbm,
                  acc_out,
                  acc_sh,
                  srcv, dstv, adjv, rows, sem):
    c = lax.axis_index("c")
    s = lax.axis_index("s")
    wid = c * NS + s

    z16 = jnp.zeros((16,), jnp.float32)

    def _zero_bufs(r, _):
        for k in range(D // 16):
            rows[r, pl.ds(k * 16, 16)] = z16
        return 0
    lax.fori_loop(0, CHUNK, _zero_bufs, 0)

    row0 = s * ROWS_PER_TILE
    for k in range(ROWS_PER_TILE // ZROWS):
        pltpu.sync_copy(rows, acc_sh.at[pl.ds(row0 + k * ZROWS, ZROWS)])

    plsc.subcore_barrier()

    ebase = wid * EPW

    def _chunk(ci, _):
        base = ebase + ci * CHUNK
        pltpu.sync_copy(src_hbm.at[pl.ds(base, CHUNK)], srcv)
        pltpu.sync_copy(dst_hbm.at[pl.ds(base, CHUNK)], dstv)
        pltpu.sync_copy(adj_hbm.at[pl.ds(base, CHUNK)], adjv)
        pltpu.async_copy(h_hbm.at[srcv], rows, sem).wait()

        def _scale(g, _):
            av = adjv[pl.ds(g * 16, 16)]
            for j in range(16):
                a = jnp.full((16,), av[j])
                e = g * 16 + j
                for k in range(D // 16):
                    rows[e, pl.ds(k * 16, 16)] = rows[e, pl.ds(k * 16, 16)] * a
            return 0
        lax.fori_loop(0, CHUNK // 16, _scale, 0)

        pltpu.sync_copy(rows, acc_sh.at[dstv], add=True)
        return 0

    lax.fori_loop(0, N_CHUNKS, _chunk, 0)

    plsc.subcore_barrier()

    for k in range(ROWS_PER_TILE // ZROWS):
        r = row0 + k * ZROWS
        pltpu.sync_copy(acc_sh.at[pl.ds(r, ZROWS)], rows)
        pltpu.sync_copy(rows, acc_out.at[c, pl.ds(r, ZROWS)])


def _sc_feat(h, src, dst, adj):
    mesh = plsc.VectorSubcoreMesh(core_axis_name="c", subcore_axis_name="s")
    f = pl.kernel(
        _sc_feat_body,
        out_type=jax.ShapeDtypeStruct((NC, N_PAD, D), jnp.float32),
        mesh=mesh,
        scratch_types=[
            pltpu.VMEM_SHARED((N_PAD, D), jnp.float32),
            pltpu.VMEM((CHUNK,), jnp.int32),
            pltpu.VMEM((CHUNK,), jnp.int32),
            pltpu.VMEM((CHUNK,), jnp.float32),
            pltpu.VMEM((CHUNK, D), jnp.float32),
            pltpu.SemaphoreType.DMA,
        ],
    )
    return f(h, src, dst, adj)


def _sc_norm_body(dst_hbm, adj_hbm, norm_out, norm_sh, dstv, adjv, normrow):
    c = lax.axis_index("c")
    s = lax.axis_index("s")
    wid = c * NS + s

    z16 = jnp.zeros((16,), jnp.float32)

    def _zero_bufs(r, _):
        for k in range(D // 16):
            normrow[r, pl.ds(k * 16, 16)] = z16
        return 0
    lax.fori_loop(0, CHUNK, _zero_bufs, 0)

    row0 = s * ROWS_PER_TILE
    for k in range(ROWS_PER_TILE // ZROWS):
        pltpu.sync_copy(normrow, norm_sh.at[pl.ds(row0 + k * ZROWS, ZROWS)])

    plsc.subcore_barrier()

    ebase = wid * EPW

    def _chunk(ci, _):
        base = ebase + ci * CHUNK
        pltpu.sync_copy(dst_hbm.at[pl.ds(base, CHUNK)], dstv)
        pltpu.sync_copy(adj_hbm.at[pl.ds(base, CHUNK)], adjv)

        def _fill(g, _):
            av = adjv[pl.ds(g * 16, 16)]
            for j in range(16):
                a = jnp.full((16,), av[j])
                e = g * 16 + j
                for k in range(D // 16):
                    normrow[e, pl.ds(k * 16, 16)] = a
            return 0
        lax.fori_loop(0, CHUNK // 16, _fill, 0)

        pltpu.sync_copy(normrow, norm_sh.at[dstv], add=True)
        return 0

    lax.fori_loop(0, N_CHUNKS, _chunk, 0)

    plsc.subcore_barrier()

    for k in range(ROWS_PER_TILE // ZROWS):
        r = row0 + k * ZROWS
        pltpu.sync_copy(norm_sh.at[pl.ds(r, ZROWS)], normrow)
        pltpu.sync_copy(normrow, norm_out.at[c, pl.ds(r, ZROWS)])


def _sc_norm(dst, adj):
    mesh = plsc.VectorSubcoreMesh(core_axis_name="c", subcore_axis_name="s")
    f = pl.kernel(
        _sc_norm_body,
        out_type=jax.ShapeDtypeStruct((NC, N_PAD, D), jnp.float32),
        mesh=mesh,
        scratch_types=[
            pltpu.VMEM_SHARED((N_PAD, D), jnp.float32),
            pltpu.VMEM((CHUNK,), jnp.int32),
            pltpu.VMEM((CHUNK,), jnp.float32),
            pltpu.VMEM((CHUNK, D), jnp.float32),
        ],
    )
    return f(dst, adj)


def kernel(node_feat, edge_index, adj_values, W, b, W_self, b_self):
    x = node_feat[0]
    dst = edge_index[0].astype(jnp.int32)
    src = edge_index[1].astype(jnp.int32)
    pad = E_PAD - N_EDGES
    src_p = jnp.concatenate([src, jnp.zeros((pad,), jnp.int32)])
    dst_p = jnp.concatenate([dst, jnp.zeros((pad,), jnp.int32)])
    adj_p = jnp.concatenate([adj_values, jnp.zeros((pad,), jnp.float32)])
    x_pad = jnp.concatenate(
        [x, jnp.zeros((N_PAD - N_NODES, D), jnp.float32)], axis=0)

    h = _tc_linear(x, W, b)
    acc = _sc_feat(h, src_p, dst_p, adj_p)
    normp = _sc_norm(dst_p, adj_p)
    out = _tc_combine(acc[0], acc[1], normp[0], normp[1], x_pad, W_self, b_self)
    return out[:N_NODES][None]


# speedup vs baseline: 3.4911x; 1.3078x over previous
import jax
import jax.numpy as jnp
from jax import lax
from jax.experimental import pallas as pl
from jax.experimental.pallas import tpu as pltpu
from jax.experimental.pallas import tpu_sc as plsc

N_NODES = 10000
D = 128
N_EDGES = 320000

NC = 2
NS = 16
NW = NC * NS
CHUNK = 128
FCHUNK = 64
FN_CHUNKS = 10240 // FCHUNK
E_PAD = 327680
EPW = E_PAD // NW
N_CHUNKS = EPW // CHUNK
N_PAD = 10240
ROWS_PER_TILE = N_PAD // NS
ZROWS = 128
BLK = 1024


def _linear_body(x_ref, w_ref, b_ref, o_ref):
    o_ref[...] = lax.dot_general(
        x_ref[...], w_ref[...], (((1,), (1,)), ((), ())),
        preferred_element_type=jnp.float32) + b_ref[...]


def _tc_linear(x, W, b):
    return pl.pallas_call(
        _linear_body,
        grid=(10,),
        in_specs=[
            pl.BlockSpec((1000, D), lambda i: (i, 0)),
            pl.BlockSpec((D, D), lambda i: (0, 0)),
            pl.BlockSpec((1, D), lambda i: (0, 0)),
        ],
        out_specs=pl.BlockSpec((1000, D), lambda i: (i, 0)),
        out_shape=jax.ShapeDtypeStruct((N_NODES, D), jnp.float32),
    )(x, W, b.reshape(1, D))


def _combine_body(a0_ref, a1_ref, n0_ref, n1_ref, x_ref, w_ref, b_ref, o_ref):
    norm = n0_ref[...][:, 0:1] + n1_ref[...][:, 0:1]
    acc = a0_ref[...] + a1_ref[...]
    selfh = lax.dot_general(
        x_ref[...], w_ref[...], (((1,), (1,)), ((), ())),
        preferred_element_type=jnp.float32) + b_ref[...]
    o_ref[...] = acc / norm + selfh


def _tc_combine(a0, a1, n0, n1, x_pad, W_self, b_self):
    return pl.pallas_call(
        _combine_body,
        grid=(10,),
        in_specs=[
            pl.BlockSpec((BLK, D), lambda i: (i, 0)),
            pl.BlockSpec((BLK, D), lambda i: (i, 0)),
            pl.BlockSpec((BLK, D), lambda i: (i, 0)),
            pl.BlockSpec((BLK, D), lambda i: (i, 0)),
            pl.BlockSpec((BLK, D), lambda i: (i, 0)),
            pl.BlockSpec((D, D), lambda i: (0, 0)),
            pl.BlockSpec((1, D), lambda i: (0, 0)),
        ],
        out_specs=pl.BlockSpec((BLK, D), lambda i: (i, 0)),
        out_shape=jax.ShapeDtypeStruct((N_PAD, D), jnp.float32),
    )(a0, a1, n0, n1, x_pad, W_self, b_self.reshape(1, D))


def _sc_feat_body(h_hbm, src_hbm, dst_hbm, adj_hbm,
                  acc_out,
                  acc_sh,
                  srcv0, dstv0, adjv0, srcv1, dstv1, adjv1,
                  rows0, rows1, semi0, semi1, semg0, semg1):
    c = lax.axis_index("c")
    s = lax.axis_index("s")
    wid = c * NS + s
    ebase = wid * EPW

    def idx_start(ci, sv, dv, av, sem):
        base = ebase + ci * FCHUNK
        pltpu.async_copy(src_hbm.at[pl.ds(base, FCHUNK)], sv, sem)
        pltpu.async_copy(dst_hbm.at[pl.ds(base, FCHUNK)], dv, sem)
        pltpu.async_copy(adj_hbm.at[pl.ds(base, FCHUNK)], av, sem)

    def idx_wait(sv, dv, av, sem):
        pltpu.make_async_copy(src_hbm.at[pl.ds(0, FCHUNK)], sv, sem).wait()
        pltpu.make_async_copy(dst_hbm.at[pl.ds(0, FCHUNK)], dv, sem).wait()
        pltpu.make_async_copy(adj_hbm.at[pl.ds(0, FCHUNK)], av, sem).wait()

    def gather_start(sv, rows, sem):
        pltpu.async_copy(h_hbm.at[sv], rows, sem)

    def gather_wait(sv, rows, sem):
        pltpu.make_async_copy(h_hbm.at[sv], rows, sem).wait()

    def scale(rows, av_ref):
        def _scale(g, _):
            av = av_ref[pl.ds(g * 16, 16)]
            for j in range(16):
                a = jnp.full((16,), av[j])
                e = g * 16 + j
                for k in range(D // 16):
                    rows[e, pl.ds(k * 16, 16)] = rows[e, pl.ds(k * 16, 16)] * a
            return 0
        lax.fori_loop(0, FCHUNK // 16, _scale, 0)

    z16 = jnp.zeros((16,), jnp.float32)

    def _zero_bufs(r, _):
        for k in range(D // 16):
            rows0[r, pl.ds(k * 16, 16)] = z16
            rows1[r, pl.ds(k * 16, 16)] = z16
        return 0
    lax.fori_loop(0, FCHUNK, _zero_bufs, 0)

    row0 = s * ROWS_PER_TILE
    for k in range(ROWS_PER_TILE // FCHUNK):
        pltpu.sync_copy(rows0, acc_sh.at[pl.ds(row0 + k * FCHUNK, FCHUNK)])

    plsc.subcore_barrier()

    idx_start(0, srcv0, dstv0, adjv0, semi0)
    idx_wait(srcv0, dstv0, adjv0, semi0)
    gather_start(srcv0, rows0, semg0)
    idx_start(1, srcv1, dstv1, adjv1, semi1)

    def _pair(k, _):
        c0 = 2 * k

        idx_wait(srcv1, dstv1, adjv1, semi1)
        gather_start(srcv1, rows1, semg1)
        gather_wait(srcv0, rows0, semg0)
        scale(rows0, adjv0)
        pltpu.sync_copy(rows0, acc_sh.at[dstv0], add=True)

        @pl.when(c0 + 2 < FN_CHUNKS)
        def _():
            idx_start(c0 + 2, srcv0, dstv0, adjv0, semi0)
            idx_wait(srcv0, dstv0, adjv0, semi0)
            gather_start(srcv0, rows0, semg0)

        gather_wait(srcv1, rows1, semg1)
        scale(rows1, adjv1)
        pltpu.sync_copy(rows1, acc_sh.at[dstv1], add=True)

        @pl.when(c0 + 3 < FN_CHUNKS)
        def _():
            idx_start(c0 + 3, srcv1, dstv1, adjv1, semi1)
        return 0

    lax.fori_loop(0, FN_CHUNKS // 2, _pair, 0)

    plsc.subcore_barrier()

    for k in range(ROWS_PER_TILE // FCHUNK):
        r = row0 + k * FCHUNK
        pltpu.sync_copy(acc_sh.at[pl.ds(r, FCHUNK)], rows0)
        pltpu.sync_copy(rows0, acc_out.at[c, pl.ds(r, FCHUNK)])


def _sc_feat(h, src, dst, adj):
    mesh = plsc.VectorSubcoreMesh(core_axis_name="c", subcore_axis_name="s")
    f = pl.kernel(
        _sc_feat_body,
        out_type=jax.ShapeDtypeStruct((NC, N_PAD, D), jnp.float32),
        mesh=mesh,
        scratch_types=[
            pltpu.VMEM_SHARED((N_PAD, D), jnp.float32),
            pltpu.VMEM((FCHUNK,), jnp.int32),
            pltpu.VMEM((FCHUNK,), jnp.int32),
            pltpu.VMEM((FCHUNK,), jnp.float32),
            pltpu.VMEM((FCHUNK,), jnp.int32),
            pltpu.VMEM((FCHUNK,), jnp.int32),
            pltpu.VMEM((FCHUNK,), jnp.float32),
            pltpu.VMEM((FCHUNK, D), jnp.float32),
            pltpu.VMEM((FCHUNK, D), jnp.float32),
            pltpu.SemaphoreType.DMA,
            pltpu.SemaphoreType.DMA,
            pltpu.SemaphoreType.DMA,
            pltpu.SemaphoreType.DMA,
        ],
    )
    return f(h, src, dst, adj)


def _sc_norm_body(dst_hbm, adj_hbm, norm_out, norm_sh, dstv, adjv, normrow):
    c = lax.axis_index("c")
    s = lax.axis_index("s")
    wid = c * NS + s

    z16 = jnp.zeros((16,), jnp.float32)

    def _zero_bufs(r, _):
        for k in range(D // 16):
            normrow[r, pl.ds(k * 16, 16)] = z16
        return 0
    lax.fori_loop(0, CHUNK, _zero_bufs, 0)

    row0 = s * ROWS_PER_TILE
    for k in range(ROWS_PER_TILE // ZROWS):
        pltpu.sync_copy(normrow, norm_sh.at[pl.ds(row0 + k * ZROWS, ZROWS)])

    plsc.subcore_barrier()

    ebase = wid * EPW

    def _chunk(ci, _):
        base = ebase + ci * CHUNK
        pltpu.sync_copy(dst_hbm.at[pl.ds(base, CHUNK)], dstv)
        pltpu.sync_copy(adj_hbm.at[pl.ds(base, CHUNK)], adjv)

        def _fill(g, _):
            av = adjv[pl.ds(g * 16, 16)]
            for j in range(16):
                a = jnp.full((16,), av[j])
                normrow[g * 16 + j, pl.ds(0, 16)] = a
            return 0
        lax.fori_loop(0, CHUNK // 16, _fill, 0)

        pltpu.sync_copy(normrow, norm_sh.at[dstv], add=True)
        return 0

    lax.fori_loop(0, N_CHUNKS, _chunk, 0)

    plsc.subcore_barrier()

    for k in range(ROWS_PER_TILE // ZROWS):
        r = row0 + k * ZROWS
        pltpu.sync_copy(norm_sh.at[pl.ds(r, ZROWS)], normrow)
        pltpu.sync_copy(normrow, norm_out.at[c, pl.ds(r, ZROWS)])


def _sc_norm(dst, adj):
    mesh = plsc.VectorSubcoreMesh(core_axis_name="c", subcore_axis_name="s")
    f = pl.kernel(
        _sc_norm_body,
        out_type=jax.ShapeDtypeStruct((NC, N_PAD, D), jnp.float32),
        mesh=mesh,
        scratch_types=[
            pltpu.VMEM_SHARED((N_PAD, D), jnp.float32),
            pltpu.VMEM((CHUNK,), jnp.int32),
            pltpu.VMEM((CHUNK,), jnp.float32),
            pltpu.VMEM((CHUNK, D), jnp.float32),
        ],
    )
    return f(dst, adj)


def kernel(node_feat, edge_index, adj_values, W, b, W_self, b_self):
    x = node_feat[0]
    dst = edge_index[0].astype(jnp.int32)
    src = edge_index[1].astype(jnp.int32)
    pad = E_PAD - N_EDGES
    src_p = jnp.concatenate([src, jnp.zeros((pad,), jnp.int32)])
    dst_p = jnp.concatenate([dst, jnp.zeros((pad,), jnp.int32)])
    adj_p = jnp.concatenate([adj_values, jnp.zeros((pad,), jnp.float32)])
    x_pad = jnp.concatenate(
        [x, jnp.zeros((N_PAD - N_NODES, D), jnp.float32)], axis=0)

    h = _tc_linear(x, W, b)
    acc = _sc_feat(h, src_p, dst_p, adj_p)
    normp = _sc_norm(dst_p, adj_p)
    out = _tc_combine(acc[0], acc[1], normp[0], normp[1], x_pad, W_self, b_self)
    return out[:N_NODES][None]
